# Initial kernel scaffold; baseline (speedup 1.0000x reference)
import jax
import jax.numpy as jnp
from jax.experimental import pallas as pl


def kernel(params, x_component, x_pin, x_subcircuit, x_net, ei_comp_pin, ei_pin_comp, ei_sub_pin, ei_pin_sub, ei_pin_net, ei_net_pin, batch_component):
    raise NotImplementedError("write your pallas kernel here")


# speedup vs baseline: 5.7911x; 5.7911x over previous
import functools

import jax
import jax.numpy as jnp
from jax import lax
from jax.experimental import pallas as pl
from jax.experimental.pallas import tpu as pltpu
from jax.experimental.pallas import tpu_sc as plsc

N = 10000
H = 128
E = 320000
NG = 64
NCLS = 10
K = 80
NCH_FULL = 250
NCH_HALF = 125
RPT = N // 16
_EDGE_ORDER = ('comp_pin', 'pin_comp', 'sub_pin', 'pin_sub', 'pin_net', 'net_pin')

@functools.lru_cache(maxsize=None)
def _get_mesh():
    return plsc.VectorSubcoreMesh(core_axis_name="c", subcore_axis_name="s")


def _fill_zero(buf, nrows, ncols):
    zv = jnp.zeros((16,), jnp.float32)

    def body(i, c):
        for u in range(ncols // 16):
            buf[i, pl.ds(u * 16, 16)] = zv
        return c

    lax.fori_loop(0, nrows, body, 0)


def _fill_ones(buf, nrows):
    ov = jnp.ones((16,), jnp.float32)

    def body(i, c):
        buf[i, :] = ov
        return c

    lax.fori_loop(0, nrows, body, 0)


def _zero_acc_rows(acc, zbuf, tid, nrows_chunk, width_rows):
    base = tid * RPT
    for q in range(RPT // nrows_chunk):
        pltpu.sync_copy(zbuf, acc.at[pl.ds(base + q * nrows_chunk, nrows_chunk)])


def _run_edges(y_ref, src_tile, dst_tile, nwin, wch, src_v, dst_v,
               acc, rows0, rows1, sem0, sem1, ones_v=None, cntacc=None):
    npairs = wch // 2
    tail = wch % 2

    def win(w, c):
        pltpu.sync_copy(src_tile.at[w], src_v)
        pltpu.sync_copy(dst_tile.at[w], dst_v)

        def pair(p, c2):
            j0 = p * 2
            j1 = j0 + 1
            cp0 = pltpu.async_copy(y_ref.at[src_v.at[j0]], rows0, sem0)
            cp1 = pltpu.async_copy(y_ref.at[src_v.at[j1]], rows1, sem1)
            cp0.wait()
            pltpu.sync_copy(rows0, acc.at[dst_v.at[j0]], add=True)
            cp1.wait()
            pltpu.sync_copy(rows1, acc.at[dst_v.at[j1]], add=True)
            return c2

        lax.fori_loop(0, npairs, pair, 0)
        if tail:
            j = wch - 1
            pltpu.async_copy(y_ref.at[src_v.at[j]], rows0, sem0).wait()
            pltpu.sync_copy(rows0, acc.at[dst_v.at[j]], add=True)
        return c

    lax.fori_loop(0, nwin, win, 0)


def _count_edges(dst_tile, nwin, wch, dst_v, ones_v, cntacc):
    def win(w, c):
        pltpu.sync_copy(dst_tile.at[w], dst_v)

        def chunk(j, c2):
            pltpu.sync_copy(ones_v, cntacc.at[dst_v.at[j]], add=True)
            return c2

        lax.fori_loop(0, wch, chunk, 0)
        return c

    lax.fori_loop(0, nwin, win, 0)


def _flush_acc(acc, A_out, out_idx, stage, zbuf, tid, nrows_chunk):
    base = tid * RPT
    for q in range(RPT // nrows_chunk):
        sl = pl.ds(base + q * nrows_chunk, nrows_chunk)
        pltpu.sync_copy(acc.at[sl], stage)
        pltpu.sync_copy(stage, A_out.at[out_idx, sl])
        pltpu.sync_copy(zbuf, acc.at[sl])


_FL = 25
_FLC = 125


def _make_sc_rel_body(n_y, core0_rels, core1_rels, nwin, wch):
    def body(*args):
        ys = args[:n_y]
        src_all, dst_all, A_out = args[n_y:n_y + 3]
        (acc, src_v, dst_v, rows0, rows1, zbuf, stage, sem0, sem1) = args[n_y + 3:]
        cid = lax.axis_index("c")
        tid = lax.axis_index("s")
        _fill_zero(zbuf, _FL, H)
        _zero_acc_rows(acc, zbuf, tid, _FL, H)
        plsc.subcore_barrier()

        def run_core(rels):
            for r in rels:
                _run_edges(ys[r], src_all.at[r, tid], dst_all.at[r, tid],
                           nwin, wch, src_v, dst_v, acc, rows0, rows1,
                           sem0, sem1)
                plsc.subcore_barrier()
                _flush_acc(acc, A_out, r, stage, zbuf, tid, _FL)
                plsc.subcore_barrier()

        @pl.when(cid == 0)
        def _():
            run_core(core0_rels)

        @pl.when(cid == 1)
        def _():
            run_core(core1_rels)

    return body


@functools.lru_cache(maxsize=None)
def _build_sc_rel(n_y, core0_rels, core1_rels, nwin, wch):
    n_rel = len(core0_rels) + len(core1_rels)
    return pl.kernel(
        _make_sc_rel_body(n_y, core0_rels, core1_rels, nwin, wch),
        out_type=jax.ShapeDtypeStruct((n_rel, N, H), jnp.float32),
        mesh=_get_mesh(),
        compiler_params=pltpu.CompilerParams(use_tc_tiling_on_sc=False),
        scratch_types=[
            pltpu.VMEM_SHARED((N, H), jnp.float32),
            pltpu.VMEM((wch, K), jnp.int32),
            pltpu.VMEM((wch, K), jnp.int32),
            pltpu.VMEM((K, H), jnp.float32),
            pltpu.VMEM((K, H), jnp.float32),
            pltpu.VMEM((_FL, H), jnp.float32),
            pltpu.VMEM((_FL, H), jnp.float32),
            pltpu.SemaphoreType.DMA,
            pltpu.SemaphoreType.DMA,
        ])


def _sc_pass1(y0, y1, y2, y3, y4, y5, src_all, dst_all):
    return _build_sc_rel(6, (0, 1, 2), (3, 4, 5), 5, 50)(
        y0, y1, y2, y3, y4, y5, src_all, dst_all)


def _sc_pass2(y0, y1, y2, y3, src_all, dst_all):
    return _build_sc_rel(4, (0, 1), (2, 3), 5, 50)(y0, y1, y2, y3,
                                                   src_all, dst_all)


def _sc3_body(y, src_all, dst_all, A_out,
              acc, src_v, dst_v, rows0, rows1, zbuf, stage, sem0, sem1):
    cid = lax.axis_index("c")
    tid = lax.axis_index("s")
    _fill_zero(zbuf, _FL, H)
    _zero_acc_rows(acc, zbuf, tid, _FL, H)
    plsc.subcore_barrier()
    _run_edges(y, src_all.at[cid, tid], dst_all.at[cid, tid], 5, 25,
               src_v, dst_v, acc, rows0, rows1, sem0, sem1)
    plsc.subcore_barrier()
    _flush_acc(acc, A_out, cid, stage, zbuf, tid, _FL)


@functools.lru_cache(maxsize=None)
def _build_sc3():
    return pl.kernel(
        _sc3_body,
        out_type=jax.ShapeDtypeStruct((2, N, H), jnp.float32),
        mesh=_get_mesh(),
        compiler_params=pltpu.CompilerParams(use_tc_tiling_on_sc=False),
        scratch_types=[
            pltpu.VMEM_SHARED((N, H), jnp.float32),
            pltpu.VMEM((25, K), jnp.int32),
            pltpu.VMEM((25, K), jnp.int32),
            pltpu.VMEM((K, H), jnp.float32),
            pltpu.VMEM((K, H), jnp.float32),
            pltpu.VMEM((_FL, H), jnp.float32),
            pltpu.VMEM((_FL, H), jnp.float32),
            pltpu.SemaphoreType.DMA,
            pltpu.SemaphoreType.DMA,
        ])


def _sc_pass3(*args):
    return _build_sc3()(*args)


def _sc_cnt_body(dst_all, cnt_out, cntacc, dst_v, ones_v, zbuf16, stage16,
                 sem0):
    cid = lax.axis_index("c")
    tid = lax.axis_index("s")
    del sem0
    _fill_zero(zbuf16, _FLC, 16)
    _fill_ones(ones_v, K)
    for q in range(RPT // _FLC):
        pltpu.sync_copy(zbuf16,
                        cntacc.at[pl.ds(tid * RPT + q * _FLC, _FLC)])
    plsc.subcore_barrier()

    def run_core(rels):
        for r in rels:
            _count_edges(dst_all.at[r, tid], 5, 50, dst_v, ones_v, cntacc)
            plsc.subcore_barrier()
            base = tid * RPT
            for q in range(RPT // _FLC):
                sl = pl.ds(base + q * _FLC, _FLC)
                pltpu.sync_copy(cntacc.at[sl], stage16)
                pltpu.sync_copy(stage16, cnt_out.at[r, sl])
                pltpu.sync_copy(zbuf16, cntacc.at[sl])
            plsc.subcore_barrier()

    @pl.when(cid == 0)
    def _():
        run_core((0, 1, 2))

    @pl.when(cid == 1)
    def _():
        run_core((3, 4, 5))


@functools.lru_cache(maxsize=None)
def _build_sc_cnt():
    return pl.kernel(
        _sc_cnt_body,
        out_type=jax.ShapeDtypeStruct((6, N, 16), jnp.float32),
        mesh=_get_mesh(),
        compiler_params=pltpu.CompilerParams(use_tc_tiling_on_sc=False),
        scratch_types=[
            pltpu.VMEM_SHARED((N, 16), jnp.float32),
            pltpu.VMEM((50, K), jnp.int32),
            pltpu.VMEM((K, 16), jnp.float32),
            pltpu.VMEM((_FLC, 16), jnp.float32),
            pltpu.VMEM((_FLC, 16), jnp.float32),
            pltpu.SemaphoreType.DMA,
        ])


def _sc_cnt(dst_all):
    return _build_sc_cnt()(dst_all)


_B = 1000


def _full_spec(shape):
    nd = len(shape)
    return pl.BlockSpec(shape, lambda i, _nd=nd: (0,) * _nd)


def _row_spec(width, block=_B):
    return pl.BlockSpec((block, width), lambda i: (i, 0))


def _embed_block(x_ref, ne, ce, pe, is_component):
    x = x_ref[...]
    nt = x[:, 0]
    pt = jnp.maximum(x[:, 2], 0)
    oh_n = (lax.broadcasted_iota(jnp.int32, (_B, 8), 1) == nt[:, None]
            ).astype(jnp.float32)
    emb = jnp.dot(oh_n, ne[...], preferred_element_type=jnp.float32)
    if is_component:
        emb = emb + ce[0:1, :]
    else:
        ct = jnp.maximum(x[:, 1], 0)
        oh_c = (lax.broadcasted_iota(jnp.int32, (_B, 16), 1) == ct[:, None]
                ).astype(jnp.float32)
        emb = emb + jnp.dot(oh_c, ce[...], preferred_element_type=jnp.float32)
    oh_p = (lax.broadcasted_iota(jnp.int32, (_B, 16), 1) == pt[:, None]
            ).astype(jnp.float32)
    return emb + jnp.dot(oh_p, pe[...], preferred_element_type=jnp.float32)


def _tc1_body(xc, xp, xs, xn, ne, ce, pe, wl, wr, bs,
              y0, y1, y2, y3, y4, y5, zc, zp, zs, zn):
    e_c = _embed_block(xc, ne, ce, pe, True)
    e_p = _embed_block(xp, ne, ce, pe, False)
    e_s = _embed_block(xs, ne, ce, pe, False)
    e_n = _embed_block(xn, ne, ce, pe, False)
    src_of = (e_c, e_p, e_s, e_p, e_p, e_n)
    outs = (y0, y1, y2, y3, y4, y5)
    for r in range(6):
        outs[r][...] = jnp.dot(src_of[r], wl[r], preferred_element_type=jnp.float32)
    dsts = (zc, zp, zs, zn)
    embs = (e_c, e_p, e_s, e_n)
    for d in range(4):
        dsts[d][...] = jnp.dot(embs[d], wr[d],
                               preferred_element_type=jnp.float32) + bs[d, :][None, :]


_tc1 = pl.pallas_call(
    _tc1_body,
    grid=(N // _B,),
    in_specs=[_row_spec(3), _row_spec(3), _row_spec(3), _row_spec(3),
              _full_spec((8, H)), _full_spec((16, H)), _full_spec((16, H)),
              _full_spec((6, H, H)), _full_spec((4, H, H)), _full_spec((4, H))],
    out_specs=[_row_spec(H)] * 10,
    out_shape=[jax.ShapeDtypeStruct((N, H), jnp.float32)] * 10,
)


def _inv(cnt_ref, r):
    return 1.0 / jnp.maximum(cnt_ref[r][:, 0:1], 1.0)


def _tc2_body(A, cnt, zc, zp, zs, zn, wl, wr, bs,
              y0, y1, y2, y3, o_zp, o_zc):
    i0 = _inv(cnt, 0)
    i1 = _inv(cnt, 1)
    i2 = _inv(cnt, 2)
    i3 = _inv(cnt, 3)
    i4 = _inv(cnt, 4)
    i5 = _inv(cnt, 5)
    x_c = jnp.maximum(zc[...] + A[1] * i1, 0.0)
    x_p = jnp.maximum(zp[...] + A[0] * i0 + A[2] * i2 + A[5] * i5, 0.0)
    x_s = jnp.maximum(zs[...] + A[3] * i3, 0.0)
    x_n = jnp.maximum(zn[...] + A[4] * i4, 0.0)
    src_of = (x_c, x_p, x_s, x_n)
    outs = (y0, y1, y2, y3)
    for r in range(4):
        outs[r][...] = jnp.dot(src_of[r], wl[r], preferred_element_type=jnp.float32)
    o_zp[...] = jnp.dot(x_p, wr[0], preferred_element_type=jnp.float32) + bs[0, :][None, :]
    o_zc[...] = jnp.dot(x_c, wr[1], preferred_element_type=jnp.float32) + bs[1, :][None, :]


_tc2 = pl.pallas_call(
    _tc2_body,
    grid=(N // _B,),
    in_specs=[pl.BlockSpec((6, _B, H), lambda i: (0, i, 0)),
              pl.BlockSpec((6, _B, 16), lambda i: (0, i, 0)),
              _row_spec(H), _row_spec(H), _row_spec(H), _row_spec(H),
              _full_spec((4, H, H)), _full_spec((2, H, H)), _full_spec((2, H))],
    out_specs=[_row_spec(H)] * 6,
    out_shape=[jax.ShapeDtypeStruct((N, H), jnp.float32)] * 6,
)


def _tc3_body(A, cnt, zp, zc, wl, wr, bs, o_y, o_zc):
    i0 = _inv(cnt, 0)
    i1 = _inv(cnt, 1)
    i2 = _inv(cnt, 2)
    i5 = _inv(cnt, 5)
    x_p = jnp.maximum(zp[...] + A[0] * i0 + A[2] * i2 + A[3] * i5, 0.0)
    x_c = jnp.maximum(zc[...] + A[1] * i1, 0.0)
    o_y[...] = jnp.dot(x_p, wl[...], preferred_element_type=jnp.float32)
    o_zc[...] = jnp.dot(x_c, wr[...], preferred_element_type=jnp.float32) + bs[0, :][None, :]


_tc3 = pl.pallas_call(
    _tc3_body,
    grid=(N // _B,),
    in_specs=[pl.BlockSpec((4, _B, H), lambda i: (0, i, 0)),
              pl.BlockSpec((6, _B, 16), lambda i: (0, i, 0)),
              _row_spec(H), _row_spec(H),
              _full_spec((H, H)), _full_spec((H, H)), _full_spec((1, H))],
    out_specs=[_row_spec(H)] * 2,
    out_shape=[jax.ShapeDtypeStruct((N, H), jnp.float32)] * 2,
)


def _tc4_body(A, cnt, zc, batch, w1, b1, w2, b2, w3, b3, out,
              sum_acc, max_acc, cnt_acc):
    i = pl.program_id(0)

    @pl.when(i == 0)
    def _():
        sum_acc[...] = jnp.zeros((NG, H), jnp.float32)
        max_acc[...] = jnp.full((NG, H), -jnp.inf, jnp.float32)
        cnt_acc[...] = jnp.zeros((NG, H), jnp.float32)

    inv1 = _inv(cnt, 1)
    comp = jnp.maximum(zc[...] + (A[0] + A[1]) * inv1, 0.0)
    b = batch[...][:, 0]
    oh = (b[:, None] == lax.broadcasted_iota(jnp.int32, (_B, NG), 1)
          ).astype(jnp.float32)
    sum_acc[...] += lax.dot_general(oh, comp, (((0,), (0,)), ((), ())),
                                    preferred_element_type=jnp.float32)
    cnt_acc[...] += jnp.sum(oh, axis=0)[:, None]
    neg = jnp.float32(-jnp.inf)
    ms = [jnp.max(jnp.where((b == g)[:, None], comp, neg), axis=0, keepdims=True)
          for g in range(NG)]
    max_acc[...] = jnp.maximum(max_acc[...], jnp.concatenate(ms, axis=0))

    @pl.when(i == (N // _B) - 1)
    def _():
        mean = sum_acc[...] / jnp.maximum(cnt_acc[...], 1.0)
        gfeat = jnp.concatenate([mean, max_acc[...]], axis=1)
        h = jnp.maximum(jnp.dot(gfeat, w1[...], preferred_element_type=jnp.float32)
                        + b1[0, :][None, :], 0.0)
        h = jnp.maximum(jnp.dot(h, w2[...], preferred_element_type=jnp.float32)
                        + b2[0, :][None, :], 0.0)
        out[...] = jnp.dot(h, w3[...], preferred_element_type=jnp.float32) \
            + b3[0, :][None, :]


_tc4 = pl.pallas_call(
    _tc4_body,
    grid=(N // _B,),
    in_specs=[pl.BlockSpec((2, _B, H), lambda i: (0, i, 0)),
              pl.BlockSpec((6, _B, 16), lambda i: (0, i, 0)),
              _row_spec(H), _row_spec(1),
              _full_spec((2 * H, H)), _full_spec((1, H)),
              _full_spec((H, 64)), _full_spec((1, 64)),
              _full_spec((64, NCLS)), _full_spec((1, NCLS))],
    out_specs=pl.BlockSpec((NG, NCLS), lambda i: (0, 0)),
    out_shape=jax.ShapeDtypeStruct((NG, NCLS), jnp.float32),
    scratch_shapes=[pltpu.VMEM((NG, H), jnp.float32),
                    pltpu.VMEM((NG, H), jnp.float32),
                    pltpu.VMEM((NG, H), jnp.float32)],
)


def kernel(params, x_component, x_pin, x_subcircuit, x_net, ei_comp_pin,
           ei_pin_comp, ei_sub_pin, ei_pin_sub, ei_pin_net, ei_net_pin,
           batch_component):
    convs = params['convs']
    cls = params['cls']
    ne = jnp.pad(params['node_emb'], ((0, 4), (0, 0)))
    ce = jnp.pad(params['comp_emb'], ((0, 7), (0, 0)))
    pe = jnp.pad(params['pin_emb'], ((0, 3), (0, 0)))

    def wsum(layer, names, key):
        s = layer[names[0]][key]
        for n in names[1:]:
            s = s + layer[n][key]
        return s

    wl1 = jnp.stack([convs[0][n]['W_l'] for n in _EDGE_ORDER])
    wr1 = jnp.stack([convs[0]['pin_comp']['W_r'],
                     wsum(convs[0], ('comp_pin', 'sub_pin', 'net_pin'), 'W_r'),
                     convs[0]['pin_sub']['W_r'],
                     convs[0]['pin_net']['W_r']])
    b1 = jnp.stack([convs[0]['pin_comp']['b_l'],
                    wsum(convs[0], ('comp_pin', 'sub_pin', 'net_pin'), 'b_l'),
                    convs[0]['pin_sub']['b_l'],
                    convs[0]['pin_net']['b_l']])

    wl2 = jnp.stack([convs[1][n]['W_l']
                     for n in ('comp_pin', 'pin_comp', 'sub_pin', 'net_pin')])
    wr2 = jnp.stack([wsum(convs[1], ('comp_pin', 'sub_pin', 'net_pin'), 'W_r'),
                     convs[1]['pin_comp']['W_r']])
    b2 = jnp.stack([wsum(convs[1], ('comp_pin', 'sub_pin', 'net_pin'), 'b_l'),
                    convs[1]['pin_comp']['b_l']])

    wl3 = convs[2]['pin_comp']['W_l']
    wr3 = convs[2]['pin_comp']['W_r']
    b3 = convs[2]['pin_comp']['b_l'][None, :]

    eis = {'comp_pin': ei_comp_pin, 'pin_comp': ei_pin_comp,
           'sub_pin': ei_sub_pin, 'pin_sub': ei_pin_sub,
           'pin_net': ei_pin_net, 'net_pin': ei_net_pin}
    src_all1 = jnp.stack([eis[n][0].reshape(16, 5, 50, K) for n in _EDGE_ORDER])
    dst_all1 = jnp.stack([eis[n][1].reshape(16, 5, 50, K) for n in _EDGE_ORDER])
    rel2 = ('comp_pin', 'pin_comp', 'sub_pin', 'net_pin')
    src_all2 = jnp.stack([eis[n][0].reshape(16, 5, 50, K) for n in rel2])
    dst_all2 = jnp.stack([eis[n][1].reshape(16, 5, 50, K) for n in rel2])
    src3 = ei_pin_comp[0].reshape(2, 16, 5, 25, K)
    dst3 = ei_pin_comp[1].reshape(2, 16, 5, 25, K)

    cnt = _sc_cnt(dst_all1)
    y10, y11, y12, y13, y14, y15, z1c, z1p, z1s, z1n = _tc1(
        x_component, x_pin, x_subcircuit, x_net, ne, ce, pe, wl1, wr1, b1)
    A1 = _sc_pass1(y10, y11, y12, y13, y14, y15, src_all1, dst_all1)
    y20, y21, y22, y23, z2p, z2c = _tc2(A1, cnt, z1c, z1p, z1s, z1n, wl2, wr2, b2)
    A2 = _sc_pass2(y20, y21, y22, y23, src_all2, dst_all2)
    y3, z3c = _tc3(A2, cnt, z2p, z2c, wl3, wr3, b3)
    A3 = _sc_pass3(y3, src3, dst3)
    out = _tc4(A3, cnt, z3c, batch_component.reshape(N, 1),
               cls[0]['W'], cls[0]['b'][None, :],
               cls[1]['W'], cls[1]['b'][None, :],
               cls[2]['W'], cls[2]['b'][None, :])
    return out
